# baseline (speedup 1.0000x reference)
import jax
import jax.numpy as jnp
from jax.experimental import pallas as pl
from jax.experimental.pallas import tpu as pltpu
from jax.experimental.pallas import tpu_sc as plsc

_LP = 256
_GATHER_WINDOW = 128
_T = 2048
_NBUF = 8


def _sc_gather(emb, idx2d):
    D = emb.shape[1]
    mesh = plsc.VectorSubcoreMesh(core_axis_name="c", subcore_axis_name="s")

    @pl.kernel(out_type=jax.ShapeDtypeStruct((_LP, D), emb.dtype), mesh=mesh)
    def gather_kernel(emb_hbm, idx_hbm, out_hbm):
        def body(i_vmem, o_vmem):
            pltpu.sync_copy(emb_hbm.at[i_vmem.at[0]], o_vmem)

        pltpu.emit_pipeline(
            body,
            grid=(_LP // _GATHER_WINDOW,),
            in_specs=[pl.BlockSpec((1, _GATHER_WINDOW), lambda i: (0, i))],
            out_specs=[pl.BlockSpec((_GATHER_WINDOW, D), lambda i: (i, 0))],
            core_axis_name="s",
            dimension_semantics=(pltpu.PARALLEL,),
        )(idx_hbm, out_hbm)

    return gather_kernel(emb, idx2d)


def _mlp_logits_lse(gathered, L, W1, b1r, W2, b2r):
    LP, D = gathered.shape
    H = W1.shape[1]
    V = W2.shape[1]
    T = _T
    nt_full = V // T
    tail = V - nt_full * T
    nt = nt_full + (1 if tail else 0)

    nq = 4
    rq = H // nq

    def kfn(g_ref, w1_ref, b1_ref, w2_hbm, w2t_ref, b2_ref, out_ref, lse_ref,
            buf, h_ref, m_ref, s_ref, sems):
        j = pl.program_id(0)

        def sub_dma(jj, q):
            return pltpu.make_async_copy(
                w2_hbm.at[pl.ds(q * rq, rq), pl.ds(jj * T, T)],
                buf.at[jax.lax.rem(jj, _NBUF), pl.ds(q * rq, rq)],
                sems.at[q, jax.lax.rem(jj, _NBUF)])

        def issue(jj):
            @pl.when(jj < nt_full)
            def _():
                for q in range(nq):
                    sub_dma(jj, q).start()

        @pl.when(j == 0)
        def _():
            lane = jax.lax.broadcasted_iota(jnp.int32, (1, LP), 1)
            maskr = (lane < L).astype(jnp.float32)
            embr = jnp.dot(maskr, g_ref[...],
                           preferred_element_type=jnp.float32)
            hr = jnp.dot(embr, w1_ref[...],
                         preferred_element_type=jnp.float32) + b1_ref[...]
            hr = jnp.maximum(hr, 0.0)
            h_ref[...] = jnp.transpose(hr, (1, 0))
            m_ref[...] = jnp.full((1, 1), -jnp.inf, jnp.float32)
            s_ref[...] = jnp.zeros((1, 1), jnp.float32)
            for p in range(_NBUF - 1):
                issue(p)

        issue(j + _NBUF - 1)

        @pl.when(j < nt_full)
        def _():
            for q in range(nq):
                sub_dma(j, q).wait()

        def compute_tile(read, is_tail):
            acc = None
            for k8 in range(H // 8):
                part = h_ref[k8 * 8:(k8 + 1) * 8, :] * read(k8)
                acc = part if acc is None else acc + part
            t = jnp.sum(acc, axis=0, keepdims=True) + b2_ref[...]
            if is_tail:
                col = j * T + jax.lax.broadcasted_iota(jnp.int32, (1, T), 1)
                t = jnp.where(col < V, t, -jnp.inf)
            out_ref[...] = t

            m_old = m_ref[...]
            tmax = jnp.max(t, axis=1, keepdims=True)
            m_new = jnp.maximum(m_old, tmax)
            s_ref[...] = (s_ref[...] * jnp.exp(m_old - m_new)
                          + jnp.sum(jnp.exp(t - m_new), axis=1,
                                    keepdims=True))
            m_ref[...] = m_new

        @pl.when(j < nt_full)
        def _():
            b = jax.lax.rem(j, _NBUF)
            compute_tile(lambda k8: buf[b, k8 * 8:(k8 + 1) * 8, :], False)

        if tail:
            @pl.when(j == nt_full)
            def _():
                compute_tile(lambda k8: w2t_ref[k8 * 8:(k8 + 1) * 8, :],
                             True)

        @pl.when(j == nt - 1)
        def _():
            lse_ref[...] = m_ref[...] + jnp.log(s_ref[...])

    return pl.pallas_call(
        kfn,
        grid=(nt,),
        in_specs=[
            pl.BlockSpec((LP, D), lambda j: (0, 0)),
            pl.BlockSpec((D, H), lambda j: (0, 0)),
            pl.BlockSpec((1, H), lambda j: (0, 0)),
            pl.BlockSpec(memory_space=pl.ANY),
            pl.BlockSpec((H, T), lambda j: (0, nt - 1)),
            pl.BlockSpec((1, T), lambda j: (0, j)),
        ],
        out_specs=[
            pl.BlockSpec((1, T), lambda j: (0, j)),
            pl.BlockSpec((1, 1), lambda j: (0, 0)),
        ],
        out_shape=[
            jax.ShapeDtypeStruct((1, V), jnp.float32),
            jax.ShapeDtypeStruct((1, 1), jnp.float32),
        ],
        scratch_shapes=[
            pltpu.VMEM((_NBUF, H, T), jnp.float32),
            pltpu.VMEM((H, 1), jnp.float32),
            pltpu.VMEM((1, 1), jnp.float32),
            pltpu.VMEM((1, 1), jnp.float32),
            pltpu.SemaphoreType.DMA((4, _NBUF)),
        ],
    )(gathered, W1, b1r, W2, W2, b2r)


def _subtract_lse(logits, lse):
    V = logits.shape[1]
    T = _T
    nt = pl.cdiv(V, T)

    def kfn(l_ref, lse_ref, o_ref):
        o_ref[...] = l_ref[...] - lse_ref[...]

    return pl.pallas_call(
        kfn,
        grid=(nt,),
        in_specs=[
            pl.BlockSpec((1, T), lambda j: (0, j)),
            pl.BlockSpec((1, 1), lambda j: (0, 0)),
        ],
        out_specs=pl.BlockSpec((1, T), lambda j: (0, j)),
        out_shape=jax.ShapeDtypeStruct((1, V), jnp.float32),
        input_output_aliases={0: 0},
    )(logits, lse)


def kernel(inputs, emb, W1, b1, W2, b2):
    L = inputs.shape[0]
    H = W1.shape[1]
    V = W2.shape[1]
    idx = jnp.zeros((_LP,), jnp.int32).at[:L].set(inputs.astype(jnp.int32))
    gathered = _sc_gather(emb, idx.reshape(1, _LP))
    logits, lse = _mlp_logits_lse(gathered, L, W1, b1.reshape(1, H),
                                  W2, b2.reshape(1, V))
    return _subtract_lse(logits, lse)


# speedup vs baseline: 1.1859x; 1.1859x over previous
import jax
import jax.numpy as jnp
from jax.experimental import pallas as pl
from jax.experimental.pallas import tpu as pltpu
from jax.experimental.pallas import tpu_sc as plsc

_LP = 256
_GATHER_WINDOW = 128
_T = 8192
_NBUF = 4


def _sc_gather(emb, idx2d):
    D = emb.shape[1]
    mesh = plsc.VectorSubcoreMesh(core_axis_name="c", subcore_axis_name="s")

    @pl.kernel(out_type=jax.ShapeDtypeStruct((_LP, D), emb.dtype), mesh=mesh)
    def gather_kernel(emb_hbm, idx_hbm, out_hbm):
        def body(i_vmem, o_vmem):
            pltpu.sync_copy(emb_hbm.at[i_vmem.at[0]], o_vmem)

        pltpu.emit_pipeline(
            body,
            grid=(_LP // _GATHER_WINDOW,),
            in_specs=[pl.BlockSpec((1, _GATHER_WINDOW), lambda i: (0, i))],
            out_specs=[pl.BlockSpec((_GATHER_WINDOW, D), lambda i: (i, 0))],
            core_axis_name="s",
            dimension_semantics=(pltpu.PARALLEL,),
        )(idx_hbm, out_hbm)

    return gather_kernel(emb, idx2d)


def _mlp_logits_lse(gathered, L, W1, b1r, W2, b2r):
    LP, D = gathered.shape
    H = W1.shape[1]
    V = W2.shape[1]
    T = _T
    nt_full = V // T
    tail = V - nt_full * T
    nt = nt_full + (1 if tail else 0)

    nq = 4
    rq = H // nq

    def kfn(g_ref, w1_ref, b1_ref, w2_hbm, w2t_ref, b2_ref, out_ref, lse_ref,
            buf, h_ref, m_ref, s_ref, sems):
        j = pl.program_id(0)

        def sub_dma(jj, q):
            return pltpu.make_async_copy(
                w2_hbm.at[pl.ds(q * rq, rq), pl.ds(jj * T, T)],
                buf.at[jax.lax.rem(jj, _NBUF), pl.ds(q * rq, rq)],
                sems.at[q, jax.lax.rem(jj, _NBUF)])

        def issue(jj):
            @pl.when(jj < nt_full)
            def _():
                for q in range(nq):
                    sub_dma(jj, q).start()

        @pl.when(j == 0)
        def _():
            lane = jax.lax.broadcasted_iota(jnp.int32, (1, LP), 1)
            maskr = (lane < L).astype(jnp.float32)
            embr = jnp.dot(maskr, g_ref[...],
                           preferred_element_type=jnp.float32)
            hr = jnp.dot(embr, w1_ref[...],
                         preferred_element_type=jnp.float32) + b1_ref[...]
            hr = jnp.maximum(hr, 0.0)
            h_ref[...] = jnp.transpose(hr, (1, 0))
            m_ref[...] = jnp.full((1, 1), -jnp.inf, jnp.float32)
            s_ref[...] = jnp.zeros((1, 1), jnp.float32)
            for p in range(_NBUF - 1):
                issue(p)

        issue(j + _NBUF - 1)

        @pl.when(j < nt_full)
        def _():
            for q in range(nq):
                sub_dma(j, q).wait()

        def compute_tile(read, is_tail):
            acc = None
            for k8 in range(H // 8):
                part = h_ref[k8 * 8:(k8 + 1) * 8, :] * read(k8)
                acc = part if acc is None else acc + part
            t = jnp.sum(acc, axis=0, keepdims=True) + b2_ref[...]
            if is_tail:
                col = j * T + jax.lax.broadcasted_iota(jnp.int32, (1, T), 1)
                t = jnp.where(col < V, t, -jnp.inf)
            out_ref[...] = t

            m_old = m_ref[...]
            tmax = jnp.max(t, axis=1, keepdims=True)
            m_new = jnp.maximum(m_old, tmax)
            s_ref[...] = (s_ref[...] * jnp.exp(m_old - m_new)
                          + jnp.sum(jnp.exp(t - m_new), axis=1,
                                    keepdims=True))
            m_ref[...] = m_new

        @pl.when(j < nt_full)
        def _():
            b = jax.lax.rem(j, _NBUF)
            compute_tile(lambda k8: buf[b, k8 * 8:(k8 + 1) * 8, :], False)

        if tail:
            @pl.when(j == nt_full)
            def _():
                compute_tile(lambda k8: w2t_ref[k8 * 8:(k8 + 1) * 8, :],
                             True)

        @pl.when(j == nt - 1)
        def _():
            lse_ref[...] = m_ref[...] + jnp.log(s_ref[...])

    return pl.pallas_call(
        kfn,
        grid=(nt,),
        in_specs=[
            pl.BlockSpec((LP, D), lambda j: (0, 0)),
            pl.BlockSpec((D, H), lambda j: (0, 0)),
            pl.BlockSpec((1, H), lambda j: (0, 0)),
            pl.BlockSpec(memory_space=pl.ANY),
            pl.BlockSpec((H, T), lambda j: (0, nt - 1)),
            pl.BlockSpec((1, T), lambda j: (0, j)),
        ],
        out_specs=[
            pl.BlockSpec((1, T), lambda j: (0, j)),
            pl.BlockSpec((1, 1), lambda j: (0, 0)),
        ],
        out_shape=[
            jax.ShapeDtypeStruct((1, V), jnp.float32),
            jax.ShapeDtypeStruct((1, 1), jnp.float32),
        ],
        scratch_shapes=[
            pltpu.VMEM((_NBUF, H, T), jnp.float32),
            pltpu.VMEM((H, 1), jnp.float32),
            pltpu.VMEM((1, 1), jnp.float32),
            pltpu.VMEM((1, 1), jnp.float32),
            pltpu.SemaphoreType.DMA((4, _NBUF)),
        ],
    )(gathered, W1, b1r, W2, W2, b2r)


def _subtract_lse(logits, lse):
    V = logits.shape[1]
    T = _T
    nt = pl.cdiv(V, T)

    def kfn(l_ref, lse_ref, o_ref):
        o_ref[...] = l_ref[...] - lse_ref[...]

    return pl.pallas_call(
        kfn,
        grid=(nt,),
        in_specs=[
            pl.BlockSpec((1, T), lambda j: (0, j)),
            pl.BlockSpec((1, 1), lambda j: (0, 0)),
        ],
        out_specs=pl.BlockSpec((1, T), lambda j: (0, j)),
        out_shape=jax.ShapeDtypeStruct((1, V), jnp.float32),
        input_output_aliases={0: 0},
    )(logits, lse)


def kernel(inputs, emb, W1, b1, W2, b2):
    L = inputs.shape[0]
    H = W1.shape[1]
    V = W2.shape[1]
    idx = jnp.zeros((_LP,), jnp.int32).at[:L].set(inputs.astype(jnp.int32))
    gathered = _sc_gather(emb, idx.reshape(1, _LP))
    logits, lse = _mlp_logits_lse(gathered, L, W1, b1.reshape(1, H),
                                  W2, b2.reshape(1, V))
    return _subtract_lse(logits, lse)
